# baseline (speedup 1.0000x reference)
import functools

import jax
import jax.numpy as jnp
from jax import lax
from jax.experimental import pallas as pl
from jax.experimental.pallas import tpu as pltpu
from jax.experimental.pallas import tpu_sc as plsc

N = 10000
NPAD = 10240
E = 320000
EPAD = 327680
D = 128
G = 64
O = 64
EPS = 1e-5

NC = 2
NS = 16
NW = NC * NS
EW = EPAD // NW
CHUNK = 128
NCHUNK = EW // CHUNK
RPT = NPAD // NS

def _deg_body(dst_hbm, ones_hbm, z_hbm, out, didx_all, ones_v, acc_sh):
    c = lax.axis_index("c")
    s = lax.axis_index("s")
    w = c * NS + s
    r0 = s * RPT
    pltpu.sync_copy(z_hbm.at[pl.ds(r0, RPT)], acc_sh.at[pl.ds(r0, RPT)])
    pltpu.sync_copy(ones_hbm, ones_v)
    pltpu.sync_copy(dst_hbm.at[pl.ds(w * NCHUNK, NCHUNK)], didx_all)
    plsc.subcore_barrier()

    def body(j, carry):
        pltpu.sync_copy(ones_v, acc_sh.at[didx_all.at[j]], add=True)
        return carry

    lax.fori_loop(0, NCHUNK, body, 0)
    plsc.subcore_barrier()
    pltpu.sync_copy(acc_sh.at[pl.ds(r0, RPT)], out.at[c, pl.ds(r0, RPT)])


IBLK = 16
NBT = EPAD // CHUNK // IBLK
B0 = 2
B1 = NBT // NS - B0
BMAX = max(B0, B1)


def _agg_body(src_hbm, dst_hbm, table_hbm, z_hbm, out,
              sidx_blk, didx_blk, rows0, rows1, acc_sh, sem0, sem1):
    c = lax.axis_index("c")
    s = lax.axis_index("s")
    r0 = s * RPT
    pltpu.sync_copy(z_hbm.at[pl.ds(r0, RPT)], acc_sh.at[pl.ds(r0, RPT)])
    plsc.subcore_barrier()

    nblk = jnp.where(c == 0, B0, B1)
    start = jnp.where(c == 0, s * B0, NS * B0 + s * B1)

    def outer(b, carry):
        @pl.when(b < nblk)
        def _():
            base = (start + b) * IBLK
            pltpu.sync_copy(src_hbm.at[pl.ds(base, IBLK)], sidx_blk)
            pltpu.sync_copy(dst_hbm.at[pl.ds(base, IBLK)], didx_blk)
            pltpu.async_copy(table_hbm.at[sidx_blk.at[0]], rows0, sem0)

            def inner(i, carry2):
                j0 = 2 * i
                j1 = j0 + 1
                pltpu.async_copy(table_hbm.at[sidx_blk.at[j1]], rows1, sem1)
                pltpu.make_async_copy(
                    table_hbm.at[sidx_blk.at[j0]], rows0, sem0).wait()
                pltpu.sync_copy(rows0, acc_sh.at[didx_blk.at[j0]], add=True)

                @pl.when(i < IBLK // 2 - 1)
                def _():
                    pltpu.async_copy(
                        table_hbm.at[sidx_blk.at[j1 + 1]], rows0, sem0)

                pltpu.make_async_copy(
                    table_hbm.at[sidx_blk.at[j1]], rows1, sem1).wait()
                pltpu.sync_copy(rows1, acc_sh.at[didx_blk.at[j1]], add=True)
                return carry2

            lax.fori_loop(0, IBLK // 2, inner, 0)
        return carry

    lax.fori_loop(0, BMAX, outer, 0)
    plsc.subcore_barrier()
    pltpu.sync_copy(acc_sh.at[pl.ds(r0, RPT)], out.at[c, pl.ds(r0, RPT)])


@functools.lru_cache(maxsize=1)
def _sc_kernels():
    mesh = plsc.VectorSubcoreMesh(
        core_axis_name="c", subcore_axis_name="s",
        num_cores=NC, num_subcores=NS)
    deg = pl.kernel(
        _deg_body,
        out_type=jax.ShapeDtypeStruct((NC, NPAD, D), jnp.float32),
        mesh=mesh,
        scratch_types=[
            pltpu.VMEM((NCHUNK, CHUNK), jnp.int32),
            pltpu.VMEM((CHUNK, D), jnp.float32),
            pltpu.VMEM_SHARED((NPAD, D), jnp.float32),
        ],
    )
    agg = pl.kernel(
        _agg_body,
        out_type=jax.ShapeDtypeStruct((NC, NPAD, D), jnp.float32),
        mesh=mesh,
        scratch_types=[
            pltpu.VMEM((IBLK, CHUNK), jnp.int32),
            pltpu.VMEM((IBLK, CHUNK), jnp.int32),
            pltpu.VMEM((CHUNK, D), jnp.float32),
            pltpu.VMEM((CHUNK, D), jnp.float32),
            pltpu.VMEM_SHARED((NPAD, D), jnp.float32),
            pltpu.SemaphoreType.DMA,
            pltpu.SemaphoreType.DMA,
        ],
    )
    return deg, agg


BR = 512
NBLK = NPAD // BR


def _mm1_body(x_ref, w_ref, deg_ref, h1p_ref, disb_ref):
    j = pl.program_id(0)
    deg = deg_ref[0, :, 0:1] + deg_ref[1, :, 0:1]
    degc = deg + 1.0
    dis = lax.rsqrt(degc)
    rows = lax.broadcasted_iota(jnp.int32, (BR, 1), 0) + j * BR
    dism = jnp.where(rows < N, dis, 0.0)
    h = jnp.dot(x_ref[...], w_ref[...], preferred_element_type=jnp.float32)
    db = jnp.broadcast_to(dism, (BR, D))
    h1p_ref[...] = h * db
    disb_ref[...] = db


def _mm2_body(acc, h1p, disb, w2, b1, g1, be1, m1, v1, h2p_ref):
    s1 = (acc[0] + acc[1] + h1p[...]) * disb[...] + b1[...]
    z1 = (s1 - m1[...]) * lax.rsqrt(v1[...] + EPS) * g1[...] + be1[...]
    z1 = jnp.maximum(z1, 0.0)
    h2p_ref[...] = jnp.dot(
        z1, w2[...], preferred_element_type=jnp.float32) * disb[...]


def _pool_body(acc, h2p, disb, b2, g2, be2, m2, v2, batch_ref, fcw, fcb,
               out_ref, sums, cnts):
    j = pl.program_id(0)
    s2 = (acc[0] + acc[1] + h2p[...]) * disb[...] + b2[...]
    z2 = (s2 - m2[...]) * lax.rsqrt(v2[...] + EPS) * g2[...] + be2[...]
    z2 = jnp.maximum(z2, 0.0)
    bcol = batch_ref[...]
    gids = lax.broadcasted_iota(jnp.int32, (1, G), 1)
    oh = (bcol == gids).astype(jnp.float32)

    @pl.when(j == 0)
    def _():
        sums[...] = jnp.zeros((G, D), jnp.float32)
        cnts[...] = jnp.zeros((G, D), jnp.float32)

    dn = (((0,), (0,)), ((), ()))
    sums[...] += lax.dot_general(oh, z2, dn,
                                 preferred_element_type=jnp.float32)
    cnts[...] += lax.dot_general(oh, jnp.ones((BR, D), jnp.float32), dn,
                                 preferred_element_type=jnp.float32)

    @pl.when(j == NBLK - 1)
    def _():
        pooled = sums[...] / jnp.maximum(cnts[...], 1.0)
        out_ref[...] = jnp.dot(
            pooled, fcw[...], preferred_element_type=jnp.float32) + fcb[...]


def _row_spec(width):
    return pl.BlockSpec((BR, width), lambda j: (j, 0))


def _prow_spec(width):
    return pl.BlockSpec((NC, BR, width), lambda j: (0, j, 0))


def _full_spec(shape):
    return pl.BlockSpec(shape, lambda j: (0, 0))


def _sds(shape):
    return jax.ShapeDtypeStruct(shape, jnp.float32)


_mm1_call = pl.pallas_call(
    _mm1_body,
    grid=(NBLK,),
    in_specs=[_row_spec(D), _full_spec((D, D)), _prow_spec(D)],
    out_specs=[_row_spec(D), _row_spec(D)],
    out_shape=[_sds((NPAD, D)), _sds((NPAD, D))],
)

_mm2_call = pl.pallas_call(
    _mm2_body,
    grid=(NBLK,),
    in_specs=[_prow_spec(D), _row_spec(D), _row_spec(D),
              _full_spec((D, D))] + [_full_spec((1, D))] * 5,
    out_specs=_row_spec(D),
    out_shape=_sds((NPAD, D)),
)

_pool_call = pl.pallas_call(
    _pool_body,
    grid=(NBLK,),
    in_specs=[_prow_spec(D), _row_spec(D), _row_spec(D)]
             + [_full_spec((1, D))] * 5
             + [_row_spec(1), _full_spec((D, O)), _full_spec((1, O))],
    out_specs=_full_spec((G, O)),
    out_shape=_sds((G, O)),
    scratch_shapes=[pltpu.VMEM((G, D), jnp.float32),
                    pltpu.VMEM((G, D), jnp.float32)],
)


def kernel(x, edge_index, batch, W1, b1, bn1_gamma, bn1_beta, bn1_mean,
           bn1_var, W2, b2, bn2_gamma, bn2_beta, bn2_mean, bn2_var, fcW, fcb):
    f32 = jnp.float32
    pad_e = jnp.full((EPAD - E,), N, jnp.int32)
    srcp = jnp.concatenate([edge_index[0], pad_e]).reshape(EPAD // CHUNK, CHUNK)
    dstp = jnp.concatenate([edge_index[1], pad_e]).reshape(EPAD // CHUNK, CHUNK)
    xp = jnp.pad(x, ((0, NPAD - N), (0, 0)))
    batchp = jnp.concatenate(
        [batch, jnp.full((NPAD - N,), G, jnp.int32)]).reshape(NPAD, 1)
    ones128 = jnp.ones((CHUNK, D), f32)
    z128 = jnp.zeros((NPAD, D), f32)

    _deg_kernel, _agg_kernel = _sc_kernels()
    degs = _deg_kernel(dstp, ones128, z128)
    h1p, disb = _mm1_call(xp, W1, degs)
    acc1 = _agg_kernel(srcp, dstp, h1p, z128)
    h2p = _mm2_call(acc1, h1p, disb, W2,
                    b1.reshape(1, D), bn1_gamma.reshape(1, D),
                    bn1_beta.reshape(1, D), bn1_mean.reshape(1, D),
                    bn1_var.reshape(1, D))
    acc2 = _agg_kernel(srcp, dstp, h2p, z128)
    out = _pool_call(acc2, h2p, disb,
                     b2.reshape(1, D), bn2_gamma.reshape(1, D),
                     bn2_beta.reshape(1, D), bn2_mean.reshape(1, D),
                     bn2_var.reshape(1, D),
                     batchp, fcW, fcb.reshape(1, O))
    return out


# speedup vs baseline: 1.0767x; 1.0767x over previous
import functools

import jax
import jax.numpy as jnp
from jax import lax
from jax.experimental import pallas as pl
from jax.experimental.pallas import tpu as pltpu
from jax.experimental.pallas import tpu_sc as plsc

N = 10000
NPAD = 10240
E = 320000
EPAD = 327680
D = 128
G = 64
O = 64
EPS = 1e-5

NC = 2
NS = 16
NW = NC * NS
EW = EPAD // NW
CHUNK = 128
NCHUNK = EW // CHUNK
RPT = NPAD // NS

def _deg_body(dst_hbm, ones_hbm, z_hbm, out, didx_all, ones_v, acc_sh):
    c = lax.axis_index("c")
    s = lax.axis_index("s")
    w = c * NS + s
    r0 = s * RPT
    pltpu.sync_copy(z_hbm.at[pl.ds(r0, RPT)], acc_sh.at[pl.ds(r0, RPT)])
    pltpu.sync_copy(ones_hbm, ones_v)
    pltpu.sync_copy(dst_hbm.at[pl.ds(w * NCHUNK, NCHUNK)], didx_all)
    plsc.subcore_barrier()

    def body(j, carry):
        pltpu.sync_copy(ones_v, acc_sh.at[didx_all.at[j]], add=True)
        return carry

    lax.fori_loop(0, NCHUNK, body, 0)
    plsc.subcore_barrier()
    pltpu.sync_copy(acc_sh.at[pl.ds(r0, RPT)], out.at[c, pl.ds(r0, RPT)])


IBLK = 40
NBT = EPAD // CHUNK // IBLK
B0 = NBT // NS // 2
B1 = NBT // NS - B0
BMAX = max(B0, B1)


def _agg_body(src_hbm, dst_hbm, table_hbm, z_hbm, out,
              sidx_blk, didx_blk, rows0, rows1, acc_sh, sem0, sem1):
    c = lax.axis_index("c")
    s = lax.axis_index("s")
    r0 = s * RPT
    pltpu.sync_copy(z_hbm.at[pl.ds(r0, RPT)], acc_sh.at[pl.ds(r0, RPT)])
    plsc.subcore_barrier()

    nblk = jnp.where(c == 0, B0, B1)
    start = jnp.where(c == 0, s * B0, NS * B0 + s * B1)

    def outer(b, carry):
        @pl.when(b < nblk)
        def _():
            base = (start + b) * IBLK
            pltpu.sync_copy(src_hbm.at[pl.ds(base, IBLK)], sidx_blk)
            pltpu.sync_copy(dst_hbm.at[pl.ds(base, IBLK)], didx_blk)
            pltpu.async_copy(table_hbm.at[sidx_blk.at[0]], rows0, sem0)

            def inner(i, carry2):
                j0 = 2 * i
                j1 = j0 + 1
                pltpu.async_copy(table_hbm.at[sidx_blk.at[j1]], rows1, sem1)
                pltpu.make_async_copy(
                    table_hbm.at[sidx_blk.at[j0]], rows0, sem0).wait()
                pltpu.sync_copy(rows0, acc_sh.at[didx_blk.at[j0]], add=True)

                @pl.when(i < IBLK // 2 - 1)
                def _():
                    pltpu.async_copy(
                        table_hbm.at[sidx_blk.at[j1 + 1]], rows0, sem0)

                pltpu.make_async_copy(
                    table_hbm.at[sidx_blk.at[j1]], rows1, sem1).wait()
                pltpu.sync_copy(rows1, acc_sh.at[didx_blk.at[j1]], add=True)
                return carry2

            lax.fori_loop(0, IBLK // 2, inner, 0)
        return carry

    lax.fori_loop(0, BMAX, outer, 0)
    plsc.subcore_barrier()
    pltpu.sync_copy(acc_sh.at[pl.ds(r0, RPT)], out.at[c, pl.ds(r0, RPT)])


@functools.lru_cache(maxsize=1)
def _sc_kernels():
    mesh = plsc.VectorSubcoreMesh(
        core_axis_name="c", subcore_axis_name="s",
        num_cores=NC, num_subcores=NS)
    deg = pl.kernel(
        _deg_body,
        out_type=jax.ShapeDtypeStruct((NC, NPAD, D), jnp.float32),
        mesh=mesh,
        scratch_types=[
            pltpu.VMEM((NCHUNK, CHUNK), jnp.int32),
            pltpu.VMEM((CHUNK, D), jnp.float32),
            pltpu.VMEM_SHARED((NPAD, D), jnp.float32),
        ],
    )
    agg = pl.kernel(
        _agg_body,
        out_type=jax.ShapeDtypeStruct((NC, NPAD, D), jnp.float32),
        mesh=mesh,
        scratch_types=[
            pltpu.VMEM((IBLK, CHUNK), jnp.int32),
            pltpu.VMEM((IBLK, CHUNK), jnp.int32),
            pltpu.VMEM((CHUNK, D), jnp.float32),
            pltpu.VMEM((CHUNK, D), jnp.float32),
            pltpu.VMEM_SHARED((NPAD, D), jnp.float32),
            pltpu.SemaphoreType.DMA,
            pltpu.SemaphoreType.DMA,
        ],
    )
    return deg, agg


BR = 512
NBLK = NPAD // BR


def _mm1_body(x_ref, w_ref, deg_ref, h1p_ref, disb_ref):
    j = pl.program_id(0)
    deg = deg_ref[0, :, 0:1] + deg_ref[1, :, 0:1]
    degc = deg + 1.0
    dis = lax.rsqrt(degc)
    rows = lax.broadcasted_iota(jnp.int32, (BR, 1), 0) + j * BR
    dism = jnp.where(rows < N, dis, 0.0)
    h = jnp.dot(x_ref[...], w_ref[...], preferred_element_type=jnp.float32)
    db = jnp.broadcast_to(dism, (BR, D))
    h1p_ref[...] = h * db
    disb_ref[...] = db


def _mm2_body(acc, h1p, disb, w2, b1, g1, be1, m1, v1, h2p_ref):
    s1 = (acc[0] + acc[1] + h1p[...]) * disb[...] + b1[...]
    z1 = (s1 - m1[...]) * lax.rsqrt(v1[...] + EPS) * g1[...] + be1[...]
    z1 = jnp.maximum(z1, 0.0)
    h2p_ref[...] = jnp.dot(
        z1, w2[...], preferred_element_type=jnp.float32) * disb[...]


def _pool_body(acc, h2p, disb, b2, g2, be2, m2, v2, batch_ref, fcw, fcb,
               out_ref, sums, cnts):
    j = pl.program_id(0)
    s2 = (acc[0] + acc[1] + h2p[...]) * disb[...] + b2[...]
    z2 = (s2 - m2[...]) * lax.rsqrt(v2[...] + EPS) * g2[...] + be2[...]
    z2 = jnp.maximum(z2, 0.0)
    bcol = batch_ref[...]
    gids = lax.broadcasted_iota(jnp.int32, (1, G), 1)
    oh = (bcol == gids).astype(jnp.float32)

    @pl.when(j == 0)
    def _():
        sums[...] = jnp.zeros((G, D), jnp.float32)
        cnts[...] = jnp.zeros((G, D), jnp.float32)

    dn = (((0,), (0,)), ((), ()))
    sums[...] += lax.dot_general(oh, z2, dn,
                                 preferred_element_type=jnp.float32)
    cnts[...] += lax.dot_general(oh, jnp.ones((BR, D), jnp.float32), dn,
                                 preferred_element_type=jnp.float32)

    @pl.when(j == NBLK - 1)
    def _():
        pooled = sums[...] / jnp.maximum(cnts[...], 1.0)
        out_ref[...] = jnp.dot(
            pooled, fcw[...], preferred_element_type=jnp.float32) + fcb[...]


def _row_spec(width):
    return pl.BlockSpec((BR, width), lambda j: (j, 0))


def _prow_spec(width):
    return pl.BlockSpec((NC, BR, width), lambda j: (0, j, 0))


def _full_spec(shape):
    return pl.BlockSpec(shape, lambda j: (0, 0))


def _sds(shape):
    return jax.ShapeDtypeStruct(shape, jnp.float32)


_mm1_call = pl.pallas_call(
    _mm1_body,
    grid=(NBLK,),
    in_specs=[_row_spec(D), _full_spec((D, D)), _prow_spec(D)],
    out_specs=[_row_spec(D), _row_spec(D)],
    out_shape=[_sds((NPAD, D)), _sds((NPAD, D))],
)

_mm2_call = pl.pallas_call(
    _mm2_body,
    grid=(NBLK,),
    in_specs=[_prow_spec(D), _row_spec(D), _row_spec(D),
              _full_spec((D, D))] + [_full_spec((1, D))] * 5,
    out_specs=_row_spec(D),
    out_shape=_sds((NPAD, D)),
)

_pool_call = pl.pallas_call(
    _pool_body,
    grid=(NBLK,),
    in_specs=[_prow_spec(D), _row_spec(D), _row_spec(D)]
             + [_full_spec((1, D))] * 5
             + [_row_spec(1), _full_spec((D, O)), _full_spec((1, O))],
    out_specs=_full_spec((G, O)),
    out_shape=_sds((G, O)),
    scratch_shapes=[pltpu.VMEM((G, D), jnp.float32),
                    pltpu.VMEM((G, D), jnp.float32)],
)


def kernel(x, edge_index, batch, W1, b1, bn1_gamma, bn1_beta, bn1_mean,
           bn1_var, W2, b2, bn2_gamma, bn2_beta, bn2_mean, bn2_var, fcW, fcb):
    f32 = jnp.float32
    pad_e = jnp.full((EPAD - E,), N, jnp.int32)
    srcp = jnp.concatenate([edge_index[0], pad_e]).reshape(EPAD // CHUNK, CHUNK)
    dstp = jnp.concatenate([edge_index[1], pad_e]).reshape(EPAD // CHUNK, CHUNK)
    xp = jnp.pad(x, ((0, NPAD - N), (0, 0)))
    batchp = jnp.concatenate(
        [batch, jnp.full((NPAD - N,), G, jnp.int32)]).reshape(NPAD, 1)
    ones128 = jnp.ones((CHUNK, D), f32)
    z128 = jnp.zeros((NPAD, D), f32)

    _deg_kernel, _agg_kernel = _sc_kernels()
    degs = _deg_kernel(dstp, ones128, z128)
    h1p, disb = _mm1_call(xp, W1, degs)
    acc1 = _agg_kernel(srcp, dstp, h1p, z128)
    h2p = _mm2_call(acc1, h1p, disb, W2,
                    b1.reshape(1, D), bn1_gamma.reshape(1, D),
                    bn1_beta.reshape(1, D), bn1_mean.reshape(1, D),
                    bn1_var.reshape(1, D))
    acc2 = _agg_kernel(srcp, dstp, h2p, z128)
    out = _pool_call(acc2, h2p, disb,
                     b2.reshape(1, D), bn2_gamma.reshape(1, D),
                     bn2_beta.reshape(1, D), bn2_mean.reshape(1, D),
                     bn2_var.reshape(1, D),
                     batchp, fcW, fcb.reshape(1, O))
    return out
